# baseline (speedup 1.0000x reference)
import jax
import jax.numpy as jnp
import numpy as np
from jax.experimental import pallas as pl
from jax.experimental.pallas import tpu as pltpu

_B, _N, _M, _T, _S = 32, 20, 400, 21, 8
_A = _N + 1
_L = _A + _M
_D, _H, _DH = 256, 8, 32
_R, _P, _F, _K = 6, 50, 80, 6
_NEG = -1e9
_SCALE = 1.0 / np.sqrt(_DH)
_DT = 0.1
_TWO_PI = 2.0 * np.pi

_INTERPRET = False


def _relu(x):
    return jnp.maximum(x, 0.0)


def _elu(x):
    return jnp.where(x > 0, x, jnp.exp(jnp.minimum(x, 0.0)) - 1.0)


def _dot(x, w):
    return jnp.dot(x, w, preferred_element_type=jnp.float32)


def _dot_t(x, y):
    return jax.lax.dot_general(x, y, (((1,), (1,)), ((), ())),
                               preferred_element_type=jnp.float32)


def _mha_heads(q, k, v, mrow):
    outs = []
    for h in range(_H):
        sl = slice(h * _DH, (h + 1) * _DH)
        lg = _dot_t(q[:, sl], k[:, sl]) * _SCALE
        lg = jnp.where(mrow > 0.5, _NEG, lg)
        m = jnp.max(lg, axis=-1, keepdims=True)
        e = jnp.exp(lg - m)
        w = e / jnp.sum(e, axis=-1, keepdims=True)
        outs.append(_dot(w, v[:, sl]))
    return jnp.concatenate(outs, axis=-1)


def _csum(x):
    n = x.shape[-1]
    s = 1
    while s < n:
        x = x + jnp.concatenate(
            [jnp.zeros((1, s), jnp.float32), x[:, :-s]], axis=1)
        s *= 2
    return x


def _body(enc_r, cur_r, rp_r, maskf_r, mapf_r, actf_r, envf_r,
          ca_wq, ca_wk, ca_wv, ca_wo,
          mm_wq, mm_wk, mm_wv, mm_wo,
          it_wq, it_wk, it_wv, it_wo,
          dl_wq, dl_wk, dl_wv, dl_wo,
          fu_w1a, fu_w1b, fu_b1, fu_w2, fu_b2,
          g_wt, g_bt, g_ws, g_bs, g_wtraj, g_btraj,
          fe_wt, fe_bt, fe_wx, fe_bx, fe_wo, fe_bo,
          r_w1, r_b1, r_w2, r_b2,
          dm_w1, dm_b1, dm_w2, dm_b2, dm_wsc, dm_bsc,
          p_w1, p_b1, p_w2, p_b2, p_w3a, p_w3s, p_b3a, p_b3s,
          mpos,
          ap_o, sc_o, plan_o):
    enc = enc_r[0]
    cur = cur_r[0]
    rp = rp_r[0]
    mrow_l = maskf_r[0]
    mrow_m = mapf_r[0]
    mrow_a = actf_r[0]
    mrow_e = envf_r[0]

    enc_agent = enc[:_A]
    enc_map = enc[_A:]

    q_ag = _dot(enc_agent, ca_wq[...])
    al = _dot(_mha_heads(q_ag, _dot(enc_map, ca_wk[...]),
                         _dot(enc_map, ca_wv[...]), mrow_m), ca_wo[...])
    aa = _dot(_mha_heads(q_ag, _dot(enc_agent, ca_wk[...]),
                         _dot(enc_agent, ca_wv[...]), mrow_a), ca_wo[...])

    inter = _relu(_dot(al, fu_w1a[...]) + _dot(aa, fu_w1b[...]) + fu_b1[...])
    inter = _dot(inter, fu_w2[...]) + fu_b2[...]

    att = _dot(_mha_heads(_dot(inter, mm_wq[...]), _dot(al, mm_wk[...]),
                          _dot(al, mm_wv[...]), mrow_a), mm_wo[...])

    k_it = _dot(enc, it_wk[...])
    v_it = _dot(enc, it_wv[...])
    for _ in range(3):
        upd = _dot(_mha_heads(_dot(att, it_wq[...]), k_it, v_it, mrow_l),
                   it_wo[...])
        att = att + upd

    ap = _dot(att, g_wt[...]) + g_bt[...]
    sc = _dot(att, g_ws[...]) + g_bs[...]
    ap_o[0] = ap
    sc_o[0] = sc

    msc = jnp.max(sc, axis=-1, keepdims=True)
    esc = jnp.exp(sc - msc)
    wmod = esc / jnp.sum(esc, axis=-1, keepdims=True)
    state_emb = _dot(cur, fe_wx[...]) + fe_bx[...]
    fut_acc = jnp.zeros((_A, _D), jnp.float32)
    for k in range(_K):
        tk = _dot(att, g_wtraj[:, k * 2 * _F:(k + 1) * 2 * _F]) \
            + g_btraj[:, k * 2 * _F:(k + 1) * 2 * _F]
        fk = _relu(_dot(tk, fe_wt[...]) + fe_bt[...] + state_emb)
        fk = _dot(fk, fe_wo[...]) + fe_bo[...]
        fut_acc = fut_acc + fk * wmod[:, k:k + 1]
    futures = fut_acc * (1.0 / _K)

    env = jnp.concatenate([futures, enc], axis=0)
    k_dl = _dot(env, dl_wk[...])
    v_dl = _dot(env, dl_wv[...])

    t = _relu(_dot(rp, r_w1[...]) + r_b1[...])
    rows = []
    pads = []
    for r_i in range(_R):
        rows.append(jnp.max(t[r_i * _P:(r_i + 1) * _P], axis=0, keepdims=True))
        chunk = jnp.abs(rp[r_i * _P:(r_i + 1) * _P])
        pads.append(jnp.max(jnp.max(chunk, axis=0, keepdims=True),
                            axis=1, keepdims=True))
    xr = jnp.concatenate(rows, axis=0)
    xr = _dot(xr, r_w2[...]) + r_b2[...]
    pad_row = jnp.concatenate(pads, axis=1)

    for _ in range(4):
        qd = _dot(xr + mpos[...], dl_wq[...])
        out = _dot(_mha_heads(qd, k_dl, v_dl, mrow_e), dl_wo[...])
        xr = xr + out
        h = _relu(_dot(xr, dm_w1[...]) + dm_b1[...])
        xr = xr + _dot(h, dm_w2[...]) + dm_b2[...]

    sc_r = _dot_t(dm_wsc[...], xr) + dm_bsc[...]
    sc_masked = jnp.where(pad_row == 0.0, _NEG, sc_r)
    idx = jnp.argmax(sc_masked, axis=-1)
    iota = jax.lax.broadcasted_iota(jnp.int32, (1, _R), 1)
    onehot = (iota == idx[:, None]).astype(jnp.float32)
    ego = _dot(onehot, xr)

    h1 = _elu(_dot(ego, p_w1[...]) + p_b1[...])
    h2 = _elu(_dot(h1, p_w2[...]) + p_b2[...])
    acc = _dot(h2, p_w3a[...]) + p_b3a[...]
    steer = _dot(h2, p_w3s[...]) + p_b3s[...]

    ego_row = cur[0:1, :]
    x0 = ego_row[0, 0]
    y0 = ego_row[0, 1]
    yaw0 = ego_row[0, 2]
    v0 = jnp.sqrt(ego_row[0, 3] ** 2 + ego_row[0, 4] ** 2)
    vel = jnp.maximum(v0 + _csum(jnp.clip(acc, -5.0, 5.0) * _DT), 0.0)
    yaw_un = yaw0 + _csum(jnp.clip(steer, -0.5, 0.5) * vel * _DT)
    q = (yaw_un * (1.0 / _TWO_PI)).astype(jnp.int32).astype(jnp.float32)
    yaw = yaw_un - q * _TWO_PI
    xs = x0 + _csum(vel * jnp.cos(yaw) * _DT)
    ys = y0 + _csum(vel * jnp.sin(yaw) * _DT)
    plan_o[0] = jnp.concatenate([xs, ys, yaw], axis=0)


def kernel(actors, encoding, mask, map_mask, actors_mask, ref_paths, params):
    f32 = jnp.float32
    cur = actors[:, :, -1].astype(f32)
    maskf = mask.astype(f32).reshape(_B, 1, _L)
    mapf = map_mask.astype(f32).reshape(_B, 1, _M)
    actf = actors_mask.astype(f32).reshape(_B, 1, _A)
    envf = jnp.concatenate([actf, maskf], axis=2)
    rp = ref_paths.reshape(_B, _R * _P, 5)

    p = params
    fu, g, fe, rr, dm, pp = (p['fusion'], p['gmm'], p['fe'], p['ref'],
                             p['dlm'], p['plan'])
    row = lambda b: b.reshape(1, -1)
    wtraj = g['Wt'].reshape(_D, _K, _F, 4)[..., :2].reshape(_D, _K * _F * 2)
    btraj = g['bt'].reshape(_K, _F, 4)[..., :2].reshape(1, _K * _F * 2)
    w3 = pp['W3'].reshape(_D, _F, 2)
    b3 = pp['b3'].reshape(_F, 2)

    weights = [
        p['ca']['Wq'], p['ca']['Wk'], p['ca']['Wv'], p['ca']['Wo'],
        p['mm']['Wq'], p['mm']['Wk'], p['mm']['Wv'], p['mm']['Wo'],
        p['it']['Wq'], p['it']['Wk'], p['it']['Wv'], p['it']['Wo'],
        p['dl']['Wq'], p['dl']['Wk'], p['dl']['Wv'], p['dl']['Wo'],
        fu['W1'][:_D], fu['W1'][_D:], row(fu['b1']), fu['W2'], row(fu['b2']),
        g['Wt'], row(g['bt']), g['Ws'], row(g['bs']), wtraj, btraj,
        fe['Wt'], row(fe['bt']), fe['Wx'], row(fe['bx']), fe['Wo'], row(fe['bo']),
        rr['W1'], row(rr['b1']), rr['W2'], row(rr['b2']),
        dm['W1'], row(dm['b1']), dm['W2'], row(dm['b2']),
        dm['Wsc'].reshape(1, _D), dm['bsc'].reshape(1, 1),
        pp['W1'], row(pp['b1']), pp['W2'], row(pp['b2']),
        w3[..., 0], w3[..., 1], row(b3[:, 0]), row(b3[:, 1]),
        p['m_pos'].reshape(1, _D),
    ]

    data = [encoding, cur, rp, maskf, mapf, actf, envf]
    data_specs = [
        pl.BlockSpec((1,) + x.shape[1:], lambda b: (b, 0, 0)) for x in data
    ]
    w_specs = [pl.BlockSpec(memory_space=pltpu.MemorySpace.VMEM)
               for _ in weights]

    out_shape = [
        jax.ShapeDtypeStruct((_B, _A, _K * _F * 4), f32),
        jax.ShapeDtypeStruct((_B, _A, _K), f32),
        jax.ShapeDtypeStruct((_B, 3, _F), f32),
    ]
    out_specs = [
        pl.BlockSpec((1, _A, _K * _F * 4), lambda b: (b, 0, 0)),
        pl.BlockSpec((1, _A, _K), lambda b: (b, 0, 0)),
        pl.BlockSpec((1, 3, _F), lambda b: (b, 0, 0)),
    ]

    ap, sc, plan = pl.pallas_call(
        _body,
        out_shape=out_shape,
        grid=(_B,),
        in_specs=data_specs + w_specs,
        out_specs=out_specs,
        compiler_params=pltpu.CompilerParams(
            dimension_semantics=("parallel",),
            vmem_limit_bytes=64 * 1024 * 1024,
        ),
        name="scband_decoder_fused",
        interpret=_INTERPRET,
    )(*data, *weights)

    agents_pred = ap.reshape(_B, _A, _K, _F, 4)
    ego_plan = plan.transpose(0, 2, 1)
    return agents_pred, sc, ego_plan


# speedup vs baseline: 1.1880x; 1.1880x over previous
import jax
import jax.numpy as jnp
import numpy as np
from jax.experimental import pallas as pl
from jax.experimental.pallas import tpu as pltpu

_B, _N, _M, _T, _S = 32, 20, 400, 21, 8
_A = _N + 1
_L = _A + _M
_D, _H, _DH = 256, 8, 32
_R, _P, _F, _K = 6, 50, 80, 6
_E = _A + _L
_NEG = -1e9
_SCALE = 1.0 / np.sqrt(_DH)
_DT = 0.1
_TWO_PI = 2.0 * np.pi

_BP = 4
_NBLK = _B // _BP
_PC = _NBLK // 2

_INTERPRET = False


def _relu(x):
    return jnp.maximum(x, 0.0)


def _elu(x):
    return jnp.where(x > 0, x, jnp.exp(jnp.minimum(x, 0.0)) - 1.0)


def _dot(x, w):
    return jnp.dot(x, w, preferred_element_type=jnp.float32)


def _dot_t(x, y):
    return jax.lax.dot_general(x, y, (((1,), (1,)), ((), ())),
                               preferred_element_type=jnp.float32)


def _mha_heads(q, k, v, mrow):
    outs = []
    for h in range(_H):
        sl = slice(h * _DH, (h + 1) * _DH)
        lg = _dot_t(q[:, sl], k[:, sl]) * _SCALE
        lg = jnp.where(mrow > 0.5, _NEG, lg)
        m = jnp.max(lg, axis=-1, keepdims=True)
        e = jnp.exp(lg - m)
        w = e / jnp.sum(e, axis=-1, keepdims=True)
        outs.append(_dot(w, v[:, sl]))
    return jnp.concatenate(outs, axis=-1)


def _csum(x):
    n, f = x.shape
    s = 1
    while s < f:
        x = x + jnp.concatenate(
            [jnp.zeros((n, s), jnp.float32), x[:, :-s]], axis=1)
        s *= 2
    return x


def _body(enc_r, cur_r, rp_r, maskf_r, mapf_r, actf_r, envf_r,
          ca_wq, ca_wk, ca_wv, ca_wo,
          mm_wq, mm_wk, mm_wv, mm_wo,
          it_wq, it_wk, it_wv, it_wo,
          dl_wq, dl_wk, dl_wv, dl_wo,
          fu_w1a, fu_w1b, fu_b1, fu_w2, fu_b2,
          g_wt, g_bt, g_ws, g_bs, g_wtraj, g_btraj,
          fe_wt, fe_bt, fe_wx, fe_bx, fe_wo, fe_bo,
          r_w1, r_b1, r_w2, r_b2,
          dm_w1, dm_b1, dm_w2, dm_b2, dm_wsc, dm_bsc,
          p_w1, p_b1, p_w2, p_b2, p_w3a, p_w3s, p_b3a, p_b3s,
          mpos,
          ap_o, sc_o, plan_o):
    enc = enc_r[0]
    cur = cur_r[0]
    rp = rp_r[0]
    maskf = maskf_r[0]
    mapf = mapf_r[0]
    actf = actf_r[0]
    envf = envf_r[0]

    def cat(xs, axis=0):
        return jnp.concatenate(xs, axis=axis)

    agents = cat([enc[i * _L:i * _L + _A] for i in range(_BP)])

    q_ag = _dot(agents, ca_wq[...])
    k_ca = _dot(enc, ca_wk[...])
    v_ca = _dot(enc, ca_wv[...])
    al_rows, aa_rows = [], []
    for i in range(_BP):
        o = i * _L
        qi = q_ag[i * _A:(i + 1) * _A]
        al_rows.append(_mha_heads(qi, k_ca[o + _A:o + _L],
                                  v_ca[o + _A:o + _L], mapf[i:i + 1]))
        aa_rows.append(_mha_heads(qi, k_ca[o:o + _A],
                                  v_ca[o:o + _A], actf[i:i + 1]))
    al = _dot(cat(al_rows), ca_wo[...])
    aa = _dot(cat(aa_rows), ca_wo[...])

    inter = _relu(_dot(al, fu_w1a[...]) + _dot(aa, fu_w1b[...]) + fu_b1[...])
    inter = _dot(inter, fu_w2[...]) + fu_b2[...]

    q_mm = _dot(inter, mm_wq[...])
    k_mm = _dot(al, mm_wk[...])
    v_mm = _dot(al, mm_wv[...])
    att = _dot(cat([
        _mha_heads(q_mm[i * _A:(i + 1) * _A], k_mm[i * _A:(i + 1) * _A],
                   v_mm[i * _A:(i + 1) * _A], actf[i:i + 1])
        for i in range(_BP)]), mm_wo[...])

    k_it = _dot(enc, it_wk[...])
    v_it = _dot(enc, it_wv[...])
    for _ in range(3):
        q_it = _dot(att, it_wq[...])
        upd = cat([
            _mha_heads(q_it[i * _A:(i + 1) * _A], k_it[i * _L:(i + 1) * _L],
                       v_it[i * _L:(i + 1) * _L], maskf[i:i + 1])
            for i in range(_BP)])
        att = att + _dot(upd, it_wo[...])

    ap = _dot(att, g_wt[...]) + g_bt[...]
    sc = _dot(att, g_ws[...]) + g_bs[...]
    ap_o[0] = ap
    sc_o[0] = sc

    msc = jnp.max(sc, axis=-1, keepdims=True)
    esc = jnp.exp(sc - msc)
    wmod = esc / jnp.sum(esc, axis=-1, keepdims=True)
    state_emb = _dot(cur, fe_wx[...]) + fe_bx[...]
    fut_acc = jnp.zeros((_BP * _A, _D), jnp.float32)
    for k in range(_K):
        tk = _dot(att, g_wtraj[:, k * 2 * _F:(k + 1) * 2 * _F]) \
            + g_btraj[:, k * 2 * _F:(k + 1) * 2 * _F]
        fk = _relu(_dot(tk, fe_wt[...]) + fe_bt[...] + state_emb)
        fk = _dot(fk, fe_wo[...]) + fe_bo[...]
        fut_acc = fut_acc + fk * wmod[:, k:k + 1]
    futures = fut_acc * (1.0 / _K)

    env = cat([x for i in range(_BP)
               for x in (futures[i * _A:(i + 1) * _A],
                         enc[i * _L:(i + 1) * _L])])
    k_dl = _dot(env, dl_wk[...])
    v_dl = _dot(env, dl_wv[...])

    t = _relu(_dot(rp, r_w1[...]) + r_b1[...])
    rows, pads = [], []
    for i in range(_BP):
        prow = []
        for r_i in range(_R):
            o = (i * _R + r_i) * _P
            rows.append(jnp.max(t[o:o + _P], axis=0, keepdims=True))
            chunk = jnp.abs(rp[o:o + _P])
            prow.append(jnp.max(jnp.max(chunk, axis=0, keepdims=True),
                                axis=1, keepdims=True))
        pads.append(cat(prow, axis=1))
    xr = cat(rows)
    xr = _dot(xr, r_w2[...]) + r_b2[...]
    pad_all = cat(pads)

    for _ in range(4):
        qd = _dot(xr + mpos[...], dl_wq[...])
        out = cat([
            _mha_heads(qd[i * _R:(i + 1) * _R], k_dl[i * _E:(i + 1) * _E],
                       v_dl[i * _E:(i + 1) * _E], envf[i:i + 1])
            for i in range(_BP)])
        xr = xr + _dot(out, dl_wo[...])
        h = _relu(_dot(xr, dm_w1[...]) + dm_b1[...])
        xr = xr + _dot(h, dm_w2[...]) + dm_b2[...]

    sc_r = cat([_dot_t(dm_wsc[...], xr[i * _R:(i + 1) * _R])
                for i in range(_BP)]) + dm_bsc[...]
    sc_masked = jnp.where(pad_all == 0.0, _NEG, sc_r)
    idx = jnp.argmax(sc_masked, axis=-1)
    iota = jax.lax.broadcasted_iota(jnp.int32, (_BP, _R), 1)
    onehot = (iota == idx[:, None]).astype(jnp.float32)
    ego = cat([_dot(onehot[i:i + 1], xr[i * _R:(i + 1) * _R])
               for i in range(_BP)])

    h1 = _elu(_dot(ego, p_w1[...]) + p_b1[...])
    h2 = _elu(_dot(h1, p_w2[...]) + p_b2[...])
    acc = _dot(h2, p_w3a[...]) + p_b3a[...]
    steer = _dot(h2, p_w3s[...]) + p_b3s[...]

    ego_rows = cat([cur[i * _A:i * _A + 1] for i in range(_BP)])
    x0 = ego_rows[:, 0:1]
    y0 = ego_rows[:, 1:2]
    yaw0 = ego_rows[:, 2:3]
    v0 = jnp.sqrt(ego_rows[:, 3:4] ** 2 + ego_rows[:, 4:5] ** 2)
    vel = jnp.maximum(v0 + _csum(jnp.clip(acc, -5.0, 5.0) * _DT), 0.0)
    yaw_un = yaw0 + _csum(jnp.clip(steer, -0.5, 0.5) * vel * _DT)
    q = (yaw_un * (1.0 / _TWO_PI)).astype(jnp.int32).astype(jnp.float32)
    yaw = yaw_un - q * _TWO_PI
    xs = x0 + _csum(vel * jnp.cos(yaw) * _DT)
    ys = y0 + _csum(vel * jnp.sin(yaw) * _DT)
    plan_o[0] = cat([xs, ys, yaw])


def kernel(actors, encoding, mask, map_mask, actors_mask, ref_paths, params):
    f32 = jnp.float32
    cur = actors[:, :, -1].astype(f32).reshape(_NBLK, _BP * _A, _S)
    enc = encoding.reshape(_NBLK, _BP * _L, _D)
    maskf = mask.astype(f32).reshape(_NBLK, _BP, _L)
    mapf = map_mask.astype(f32).reshape(_NBLK, _BP, _M)
    actf = actors_mask.astype(f32).reshape(_NBLK, _BP, _A)
    envf = jnp.concatenate([actf, maskf], axis=2)
    rp = ref_paths.reshape(_NBLK, _BP * _R * _P, 5)

    p = params
    fu, g, fe, rr, dm, pp = (p['fusion'], p['gmm'], p['fe'], p['ref'],
                             p['dlm'], p['plan'])
    row = lambda b: b.reshape(1, -1)
    wtraj = g['Wt'].reshape(_D, _K, _F, 4)[..., :2].reshape(_D, _K * _F * 2)
    btraj = g['bt'].reshape(_K, _F, 4)[..., :2].reshape(1, _K * _F * 2)
    w3 = pp['W3'].reshape(_D, _F, 2)
    b3 = pp['b3'].reshape(_F, 2)

    weights = [
        p['ca']['Wq'], p['ca']['Wk'], p['ca']['Wv'], p['ca']['Wo'],
        p['mm']['Wq'], p['mm']['Wk'], p['mm']['Wv'], p['mm']['Wo'],
        p['it']['Wq'], p['it']['Wk'], p['it']['Wv'], p['it']['Wo'],
        p['dl']['Wq'], p['dl']['Wk'], p['dl']['Wv'], p['dl']['Wo'],
        fu['W1'][:_D], fu['W1'][_D:], row(fu['b1']), fu['W2'], row(fu['b2']),
        g['Wt'], row(g['bt']), g['Ws'], row(g['bs']), wtraj, btraj,
        fe['Wt'], row(fe['bt']), fe['Wx'], row(fe['bx']), fe['Wo'], row(fe['bo']),
        rr['W1'], row(rr['b1']), rr['W2'], row(rr['b2']),
        dm['W1'], row(dm['b1']), dm['W2'], row(dm['b2']),
        dm['Wsc'].reshape(1, _D), dm['bsc'].reshape(1, 1),
        pp['W1'], row(pp['b1']), pp['W2'], row(pp['b2']),
        w3[..., 0], w3[..., 1], row(b3[:, 0]), row(b3[:, 1]),
        p['m_pos'].reshape(1, _D),
    ]

    def dmap(j):
        return (j, 0, 0)

    data = [enc, cur, rp, maskf, mapf, actf, envf]
    data_specs = [pl.BlockSpec((1,) + x.shape[1:], dmap) for x in data]
    w_specs = [pl.BlockSpec(memory_space=pltpu.MemorySpace.VMEM)
               for _ in weights]

    out_shape = [
        jax.ShapeDtypeStruct((_NBLK, _BP * _A, _K * _F * 4), f32),
        jax.ShapeDtypeStruct((_NBLK, _BP * _A, _K), f32),
        jax.ShapeDtypeStruct((_NBLK, 3 * _BP, _F), f32),
    ]
    out_specs = [
        pl.BlockSpec((1, _BP * _A, _K * _F * 4), dmap),
        pl.BlockSpec((1, _BP * _A, _K), dmap),
        pl.BlockSpec((1, 3 * _BP, _F), dmap),
    ]

    ap, sc, plan = pl.pallas_call(
        _body,
        out_shape=out_shape,
        grid=(_NBLK,),
        in_specs=data_specs + w_specs,
        out_specs=out_specs,
        compiler_params=pltpu.CompilerParams(
            dimension_semantics=("arbitrary",),
            vmem_limit_bytes=64 * 1024 * 1024,
        ),
        name="scband_decoder_fused",
        interpret=_INTERPRET,
    )(*data, *weights)

    agents_pred = ap.reshape(_B, _A, _K, _F, 4)
    scores = sc.reshape(_B, _A, _K)
    ego_plan = (plan.reshape(_NBLK, 3, _BP, _F)
                .transpose(0, 2, 3, 1).reshape(_B, _F, 3))
    return agents_pred, scores, ego_plan


# speedup vs baseline: 3.1923x; 2.6871x over previous
import jax
import jax.numpy as jnp
import numpy as np
from jax.experimental import pallas as pl
from jax.experimental.pallas import tpu as pltpu

_B, _N, _M, _T, _S = 32, 20, 400, 21, 8
_A = _N + 1
_L = _A + _M
_D, _H, _DH = 256, 8, 32
_R, _P, _F, _K = 6, 50, 80, 6
_E = _A + _L
_NEG = -1e9
_SCALE = 1.0 / np.sqrt(_DH)
_DT = 0.1
_TWO_PI = 2.0 * np.pi

_BP = 4
_NBLK = _B // _BP
_PC = _NBLK // 2

_INTERPRET = False


def _relu(x):
    return jnp.maximum(x, 0.0)


def _elu(x):
    return jnp.where(x > 0, x, jnp.exp(jnp.minimum(x, 0.0)) - 1.0)


def _dot(x, w):
    return jnp.dot(x, w, preferred_element_type=jnp.float32)


def _dot_t(x, y):
    return jax.lax.dot_general(x, y, (((1,), (1,)), ((), ())),
                               preferred_element_type=jnp.float32)


def _mha_phase(qs, ks, vs, ms):
    n = len(qs)
    hs = [slice(h * _DH, (h + 1) * _DH) for h in range(_H)]
    lg = [[_dot_t(qs[i][:, sl], ks[i][:, sl]) * _SCALE for sl in hs]
          for i in range(n)]
    lg = [[jnp.where(ms[i] > 0.5, _NEG, x) for x in lg[i]] for i in range(n)]
    mx = [[jnp.max(x, axis=-1, keepdims=True) for x in lg[i]]
          for i in range(n)]
    e = [[jnp.exp(x - m) for x, m in zip(lg[i], mx[i])] for i in range(n)]
    sm = [[jnp.sum(x, axis=-1, keepdims=True) for x in e[i]]
          for i in range(n)]
    w = [[x / s for x, s in zip(e[i], sm[i])] for i in range(n)]
    av = [[_dot(w[i][h], vs[i][:, hs[h]]) for h in range(_H)]
          for i in range(n)]
    return [jnp.concatenate(av[i], axis=-1) for i in range(n)]


def _csum(x):
    n, f = x.shape
    s = 1
    while s < f:
        x = x + jnp.concatenate(
            [jnp.zeros((n, s), jnp.float32), x[:, :-s]], axis=1)
        s *= 2
    return x


def _body(enc_r, cur_r, rp_r, maskf_r, mapf_r, actf_r, envf_r,
          ca_wq, ca_wk, ca_wv, ca_wo,
          mm_wq, mm_wk, mm_wv, mm_wo,
          it_wq, it_wk, it_wv, it_wo,
          dl_wq, dl_wk, dl_wv, dl_wo,
          fu_w1a, fu_w1b, fu_b1, fu_w2, fu_b2,
          g_wt, g_bt, g_ws, g_bs, g_wtraj, g_btraj,
          fe_wt, fe_bt, fe_wx, fe_bx, fe_wo, fe_bo,
          r_w1, r_b1, r_w2, r_b2,
          dm_w1, dm_b1, dm_w2, dm_b2, dm_wsc, dm_bsc,
          p_w1, p_b1, p_w2, p_b2, p_w3a, p_w3s, p_b3a, p_b3s,
          mpos,
          ap_o, sc_o, plan_o):
    enc = enc_r[0]
    cur = cur_r[0]
    rp = rp_r[0]
    maskf = maskf_r[0]
    mapf = mapf_r[0]
    actf = actf_r[0]
    envf = envf_r[0]

    def cat(xs, axis=0):
        return jnp.concatenate(xs, axis=axis)

    agents = cat([enc[i * _L:i * _L + _A] for i in range(_BP)])

    q_ag = _dot(agents, ca_wq[...])
    k_ca = _dot(enc, ca_wk[...])
    v_ca = _dot(enc, ca_wv[...])
    q_s = [q_ag[i * _A:(i + 1) * _A] for i in range(_BP)]
    both = _mha_phase(
        q_s + q_s,
        [k_ca[i * _L + _A:(i + 1) * _L] for i in range(_BP)]
        + [k_ca[i * _L:i * _L + _A] for i in range(_BP)],
        [v_ca[i * _L + _A:(i + 1) * _L] for i in range(_BP)]
        + [v_ca[i * _L:i * _L + _A] for i in range(_BP)],
        [mapf[i:i + 1] for i in range(_BP)]
        + [actf[i:i + 1] for i in range(_BP)])
    al = _dot(cat(both[:_BP]), ca_wo[...])
    aa = _dot(cat(both[_BP:]), ca_wo[...])

    inter = _relu(_dot(al, fu_w1a[...]) + _dot(aa, fu_w1b[...]) + fu_b1[...])
    inter = _dot(inter, fu_w2[...]) + fu_b2[...]

    q_mm = _dot(inter, mm_wq[...])
    k_mm = _dot(al, mm_wk[...])
    v_mm = _dot(al, mm_wv[...])
    att = _dot(cat(_mha_phase(
        [q_mm[i * _A:(i + 1) * _A] for i in range(_BP)],
        [k_mm[i * _A:(i + 1) * _A] for i in range(_BP)],
        [v_mm[i * _A:(i + 1) * _A] for i in range(_BP)],
        [actf[i:i + 1] for i in range(_BP)])), mm_wo[...])

    k_it = _dot(enc, it_wk[...])
    v_it = _dot(enc, it_wv[...])
    for _ in range(3):
        q_it = _dot(att, it_wq[...])
        upd = cat(_mha_phase(
            [q_it[i * _A:(i + 1) * _A] for i in range(_BP)],
            [k_it[i * _L:(i + 1) * _L] for i in range(_BP)],
            [v_it[i * _L:(i + 1) * _L] for i in range(_BP)],
            [maskf[i:i + 1] for i in range(_BP)]))
        att = att + _dot(upd, it_wo[...])

    ap = _dot(att, g_wt[...]) + g_bt[...]
    sc = _dot(att, g_ws[...]) + g_bs[...]
    ap_o[0] = ap
    sc_o[0] = sc

    msc = jnp.max(sc, axis=-1, keepdims=True)
    esc = jnp.exp(sc - msc)
    wmod = esc / jnp.sum(esc, axis=-1, keepdims=True)
    state_emb = _dot(cur, fe_wx[...]) + fe_bx[...]
    fut_acc = jnp.zeros((_BP * _A, _D), jnp.float32)
    for k in range(_K):
        tk = _dot(att, g_wtraj[:, k * 2 * _F:(k + 1) * 2 * _F]) \
            + g_btraj[:, k * 2 * _F:(k + 1) * 2 * _F]
        fk = _relu(_dot(tk, fe_wt[...]) + fe_bt[...] + state_emb)
        fk = _dot(fk, fe_wo[...]) + fe_bo[...]
        fut_acc = fut_acc + fk * wmod[:, k:k + 1]
    futures = fut_acc * (1.0 / _K)

    env = cat([x for i in range(_BP)
               for x in (futures[i * _A:(i + 1) * _A],
                         enc[i * _L:(i + 1) * _L])])
    k_dl = _dot(env, dl_wk[...])
    v_dl = _dot(env, dl_wv[...])

    t = _relu(_dot(rp, r_w1[...]) + r_b1[...])
    rows, pads = [], []
    for i in range(_BP):
        prow = []
        for r_i in range(_R):
            o = (i * _R + r_i) * _P
            rows.append(jnp.max(t[o:o + _P], axis=0, keepdims=True))
            chunk = jnp.abs(rp[o:o + _P])
            prow.append(jnp.max(jnp.max(chunk, axis=0, keepdims=True),
                                axis=1, keepdims=True))
        pads.append(cat(prow, axis=1))
    xr = cat(rows)
    xr = _dot(xr, r_w2[...]) + r_b2[...]
    pad_all = cat(pads)

    for _ in range(4):
        qd = _dot(xr + mpos[...], dl_wq[...])
        out = cat(_mha_phase(
            [qd[i * _R:(i + 1) * _R] for i in range(_BP)],
            [k_dl[i * _E:(i + 1) * _E] for i in range(_BP)],
            [v_dl[i * _E:(i + 1) * _E] for i in range(_BP)],
            [envf[i:i + 1] for i in range(_BP)]))
        xr = xr + _dot(out, dl_wo[...])
        h = _relu(_dot(xr, dm_w1[...]) + dm_b1[...])
        xr = xr + _dot(h, dm_w2[...]) + dm_b2[...]

    sc_r = cat([_dot_t(dm_wsc[...], xr[i * _R:(i + 1) * _R])
                for i in range(_BP)]) + dm_bsc[...]
    sc_masked = jnp.where(pad_all == 0.0, _NEG, sc_r)
    idx = jnp.argmax(sc_masked, axis=-1)
    iota = jax.lax.broadcasted_iota(jnp.int32, (_BP, _R), 1)
    onehot = (iota == idx[:, None]).astype(jnp.float32)
    ego = cat([_dot(onehot[i:i + 1], xr[i * _R:(i + 1) * _R])
               for i in range(_BP)])

    h1 = _elu(_dot(ego, p_w1[...]) + p_b1[...])
    h2 = _elu(_dot(h1, p_w2[...]) + p_b2[...])
    acc = _dot(h2, p_w3a[...]) + p_b3a[...]
    steer = _dot(h2, p_w3s[...]) + p_b3s[...]

    ego_rows = cat([cur[i * _A:i * _A + 1] for i in range(_BP)])
    x0 = ego_rows[:, 0:1]
    y0 = ego_rows[:, 1:2]
    yaw0 = ego_rows[:, 2:3]
    v0 = jnp.sqrt(ego_rows[:, 3:4] ** 2 + ego_rows[:, 4:5] ** 2)
    vel = jnp.maximum(v0 + _csum(jnp.clip(acc, -5.0, 5.0) * _DT), 0.0)
    yaw_un = yaw0 + _csum(jnp.clip(steer, -0.5, 0.5) * vel * _DT)
    q = (yaw_un * (1.0 / _TWO_PI)).astype(jnp.int32).astype(jnp.float32)
    yaw = yaw_un - q * _TWO_PI
    xs = x0 + _csum(vel * jnp.cos(yaw) * _DT)
    ys = y0 + _csum(vel * jnp.sin(yaw) * _DT)
    plan_o[0] = cat([xs, ys, yaw])


def kernel(actors, encoding, mask, map_mask, actors_mask, ref_paths, params):
    f32 = jnp.float32
    cur = actors[:, :, -1].astype(f32).reshape(_NBLK, _BP * _A, _S)
    enc = encoding.reshape(_NBLK, _BP * _L, _D)
    maskf = mask.astype(f32).reshape(_NBLK, _BP, _L)
    mapf = map_mask.astype(f32).reshape(_NBLK, _BP, _M)
    actf = actors_mask.astype(f32).reshape(_NBLK, _BP, _A)
    envf = jnp.concatenate([actf, maskf], axis=2)
    rp = ref_paths.reshape(_NBLK, _BP * _R * _P, 5)

    p = params
    fu, g, fe, rr, dm, pp = (p['fusion'], p['gmm'], p['fe'], p['ref'],
                             p['dlm'], p['plan'])
    row = lambda b: b.reshape(1, -1)
    wtraj = g['Wt'].reshape(_D, _K, _F, 4)[..., :2].reshape(_D, _K * _F * 2)
    btraj = g['bt'].reshape(_K, _F, 4)[..., :2].reshape(1, _K * _F * 2)
    w3 = pp['W3'].reshape(_D, _F, 2)
    b3 = pp['b3'].reshape(_F, 2)

    weights = [
        p['ca']['Wq'], p['ca']['Wk'], p['ca']['Wv'], p['ca']['Wo'],
        p['mm']['Wq'], p['mm']['Wk'], p['mm']['Wv'], p['mm']['Wo'],
        p['it']['Wq'], p['it']['Wk'], p['it']['Wv'], p['it']['Wo'],
        p['dl']['Wq'], p['dl']['Wk'], p['dl']['Wv'], p['dl']['Wo'],
        fu['W1'][:_D], fu['W1'][_D:], row(fu['b1']), fu['W2'], row(fu['b2']),
        g['Wt'], row(g['bt']), g['Ws'], row(g['bs']), wtraj, btraj,
        fe['Wt'], row(fe['bt']), fe['Wx'], row(fe['bx']), fe['Wo'], row(fe['bo']),
        rr['W1'], row(rr['b1']), rr['W2'], row(rr['b2']),
        dm['W1'], row(dm['b1']), dm['W2'], row(dm['b2']),
        dm['Wsc'].reshape(1, _D), dm['bsc'].reshape(1, 1),
        pp['W1'], row(pp['b1']), pp['W2'], row(pp['b2']),
        w3[..., 0], w3[..., 1], row(b3[:, 0]), row(b3[:, 1]),
        p['m_pos'].reshape(1, _D),
    ]

    def dmap(j):
        return (j, 0, 0)

    data = [enc, cur, rp, maskf, mapf, actf, envf]
    data_specs = [pl.BlockSpec((1,) + x.shape[1:], dmap) for x in data]
    w_specs = [pl.BlockSpec(memory_space=pltpu.MemorySpace.VMEM)
               for _ in weights]

    out_shape = [
        jax.ShapeDtypeStruct((_NBLK, _BP * _A, _K * _F * 4), f32),
        jax.ShapeDtypeStruct((_NBLK, _BP * _A, _K), f32),
        jax.ShapeDtypeStruct((_NBLK, 3 * _BP, _F), f32),
    ]
    out_specs = [
        pl.BlockSpec((1, _BP * _A, _K * _F * 4), dmap),
        pl.BlockSpec((1, _BP * _A, _K), dmap),
        pl.BlockSpec((1, 3 * _BP, _F), dmap),
    ]

    ap, sc, plan = pl.pallas_call(
        _body,
        out_shape=out_shape,
        grid=(_NBLK,),
        in_specs=data_specs + w_specs,
        out_specs=out_specs,
        compiler_params=pltpu.CompilerParams(
            dimension_semantics=("arbitrary",),
            vmem_limit_bytes=64 * 1024 * 1024,
        ),
        name="scband_decoder_fused",
        interpret=_INTERPRET,
    )(*data, *weights)

    agents_pred = ap.reshape(_B, _A, _K, _F, 4)
    scores = sc.reshape(_B, _A, _K)
    ego_plan = (plan.reshape(_NBLK, 3, _BP, _F)
                .transpose(0, 2, 3, 1).reshape(_B, _F, 3))
    return agents_pred, scores, ego_plan
